# Initial kernel scaffold; baseline (speedup 1.0000x reference)
import jax
import jax.numpy as jnp
from jax.experimental import pallas as pl


def kernel(d, table):
    raise NotImplementedError("write your pallas kernel here")


# speedup vs baseline: 1.0951x; 1.0951x over previous
import functools

import jax
import jax.numpy as jnp
from jax import lax
from jax.experimental import pallas as pl
from jax.experimental.pallas import tpu as pltpu
from jax.experimental.pallas import tpu_sc as plsc

_B = 16384 * 50
_D = 32
_NC = 2
_NS = 16
_NW = _NC * _NS
_BPW = _B // _NW
_IDXW = 128
_CHUNK = 1024
_NSUB = _CHUNK // _IDXW
_NCHUNK = _BPW // _CHUNK

_mesh = plsc.VectorSubcoreMesh(core_axis_name="c", subcore_axis_name="s")


@functools.partial(
    pl.kernel,
    mesh=_mesh,
    out_type=jax.ShapeDtypeStruct((_B, _D), jnp.float32),
    scratch_types=[
        pltpu.VMEM((_NSUB, _IDXW), jnp.int32),
        pltpu.VMEM((_CHUNK, _D), jnp.float32),
        pltpu.SemaphoreType.DMA,
    ],
    compiler_params=pltpu.CompilerParams(use_tc_tiling_on_sc=False),
)
def _gather(idx_hbm, table_hbm, out_hbm, idx_v, rows_v, sem):
    wid = lax.axis_index("s") * _NC + lax.axis_index("c")
    base = wid * _BPW

    @pl.loop(0, _NCHUNK)
    def _chunk(g):
        off = base + g * _CHUNK
        row0 = pl.multiple_of(off // _IDXW, 8)
        pltpu.sync_copy(idx_hbm.at[pl.ds(row0, _NSUB)], idx_v)
        copies = [
            pltpu.async_copy(
                table_hbm.at[idx_v.at[j]],
                rows_v.at[pl.ds(j * _IDXW, _IDXW)],
                sem,
            )
            for j in range(_NSUB)
        ]
        for cp in copies:
            cp.wait()
        pltpu.sync_copy(rows_v, out_hbm.at[pl.ds(off, _CHUNK)])


def kernel(d, table):
    idx = d.reshape(-1).astype(jnp.int32).reshape(-1, _IDXW)
    out = _gather(idx, table)
    return out.reshape(d.shape[0], d.shape[1], _D)


# speedup vs baseline: 1.4047x; 1.2828x over previous
import functools

import jax
import jax.numpy as jnp
from jax import lax
from jax.experimental import pallas as pl
from jax.experimental.pallas import tpu as pltpu
from jax.experimental.pallas import tpu_sc as plsc

_B = 16384
_H = 50
_D = 32
_V = 1000000
_N = _B * _H
_NC = 2
_NS = 16
_NW = _NC * _NS
_NPW = _N // _NW
_IDXW = 128
_CHUNK = 1024
_NSUB = _CHUNK // _IDXW
_NTASK = _NPW // _CHUNK
_IROWS = _NPW // _IDXW

_mesh = plsc.VectorSubcoreMesh(core_axis_name="c", subcore_axis_name="s")


@functools.partial(
    pl.kernel,
    mesh=_mesh,
    out_type=jax.ShapeDtypeStruct((_H, _D, _B), jnp.float32),
    scratch_types=[
        pltpu.VMEM((_IROWS, _IDXW), jnp.int32),
        pltpu.VMEM((_CHUNK, _D), jnp.float32),
        pltpu.VMEM((_D, _CHUNK), jnp.float32),
        pltpu.SemaphoreType.DMA,
        pltpu.SemaphoreType.DMA,
    ],
    compiler_params=pltpu.CompilerParams(
        use_tc_tiling_on_sc=False, needs_layout_passes=False
    ),
)
def _gather_t(idx_hbm, table_hbm, out_hbm, idx_v, rows_v, rowst_v, gsem, osem):
    wid = lax.axis_index("s") * _NC + lax.axis_index("c")
    irow0 = pl.multiple_of(wid * _IROWS, 8)
    pltpu.sync_copy(idx_hbm.at[pl.ds(irow0, _IROWS)], idx_v)

    @pl.loop(0, _NTASK)
    def _task(k):
        flat = wid * _NPW + k * _CHUNK
        h = flat // _B
        b0 = pl.multiple_of(flat % _B, _CHUNK)
        gathers = [
            pltpu.async_copy(
                table_hbm.at[idx_v.at[k * _NSUB + j]],
                rows_v.at[pl.ds(j * _IDXW, _IDXW)],
                gsem,
            )
            for j in range(_NSUB)
        ]
        for cp in gathers:
            cp.wait()

        @pl.loop(0, _CHUNK // 16)
        def _txp(j):
            row16 = j * 16 + lax.iota(jnp.int32, 16)
            for e in range(_D):
                col16 = jnp.full((16,), e, jnp.int32)
                rowst_v[e, pl.ds(j * 16, 16)] = plsc.load_gather(
                    rows_v, [row16, col16]
                )

        pltpu.async_copy(
            rowst_v,
            out_hbm.at[h, pl.ds(0, _D), pl.ds(b0, _CHUNK)],
            osem,
        ).wait()


def kernel(d, table):
    idx = d.T.reshape(_IROWS * _NW, _IDXW).astype(jnp.int32)
    out_t = _gather_t(idx, table)
    return jnp.transpose(out_t, (2, 0, 1))


# speedup vs baseline: 2.0514x; 1.4604x over previous
import functools

import jax
import jax.numpy as jnp
from jax import lax
from jax.experimental import pallas as pl
from jax.experimental.pallas import tpu as pltpu
from jax.experimental.pallas import tpu_sc as plsc

_B = 16384
_H = 50
_D = 32
_V = 1000000
_N = _B * _H
_NC = 2
_NS = 16
_NW = _NC * _NS
_NPW = _N // _NW
_IDXW = 128
_CHUNK = 1024
_NSUB = _CHUNK // _IDXW
_NTASK = _NPW // _CHUNK
_IROWS = _NPW // _IDXW

_mesh = plsc.VectorSubcoreMesh(core_axis_name="c", subcore_axis_name="s")


@functools.partial(
    pl.kernel,
    mesh=_mesh,
    out_type=jax.ShapeDtypeStruct((_H, _D, _B), jnp.float32),
    scratch_types=[
        pltpu.VMEM((_IROWS, _IDXW), jnp.int32),
        pltpu.VMEM((_CHUNK, _D), jnp.float32),
        pltpu.VMEM((_D, _CHUNK), jnp.float32),
        pltpu.SemaphoreType.DMA,
        pltpu.SemaphoreType.DMA,
    ],
    compiler_params=pltpu.CompilerParams(
        use_tc_tiling_on_sc=False, needs_layout_passes=False
    ),
)
def _gather_t(idx_hbm, table_hbm, out_hbm, idx_v, rows_v, rowst_v, gsem, osem):
    wid = lax.axis_index("s") * _NC + lax.axis_index("c")
    irow0 = pl.multiple_of(wid * _IROWS, 8)
    pltpu.sync_copy(idx_hbm.at[pl.ds(irow0, _IROWS)], idx_v)

    @pl.loop(0, _NTASK)
    def _task(k):
        flat = wid * _NPW + k * _CHUNK
        h = flat // _B
        b0 = pl.multiple_of(flat % _B, _CHUNK)
        gathers = [
            pltpu.async_copy(
                table_hbm.at[idx_v.at[k * _NSUB + j]],
                rows_v.at[pl.ds(j * _IDXW, _IDXW)],
                gsem,
            )
            for j in range(_NSUB)
        ]
        for cp in gathers:
            cp.wait()

        @pl.loop(0, _CHUNK // 16)
        def _txp(j):
            lane16 = lax.iota(jnp.int32, 16)
            row16 = j * 16 + lane16
            for e in range(_D):
                col16 = (e + lane16) & (_D - 1)
                vals = plsc.load_gather(rows_v, [row16, col16])
                plsc.store_scatter(rowst_v, [col16, row16], vals)

        pltpu.async_copy(
            rowst_v,
            out_hbm.at[h, pl.ds(0, _D), pl.ds(b0, _CHUNK)],
            osem,
        ).wait()


def kernel(d, table):
    idx = d.T.reshape(_IROWS * _NW, _IDXW).astype(jnp.int32)
    out_t = _gather_t(idx, table)
    return jnp.transpose(out_t, (2, 0, 1))


# speedup vs baseline: 2.2506x; 1.0971x over previous
import functools

import jax
import jax.numpy as jnp
from jax import lax
from jax.experimental import pallas as pl
from jax.experimental.pallas import tpu as pltpu
from jax.experimental.pallas import tpu_sc as plsc

_B = 16384
_H = 50
_D = 32
_N = _B * _H
_NC = 2
_NS = 16
_NW = _NC * _NS
_NPW = _N // _NW
_IDXW = 128
_CHUNK = 512
_NSUB = _CHUNK // _IDXW
_NTASK = _NPW // _CHUNK
_IROWS = _NPW // _IDXW

_mesh = plsc.VectorSubcoreMesh(core_axis_name="c", subcore_axis_name="s")


@functools.partial(
    pl.kernel,
    mesh=_mesh,
    out_type=jax.ShapeDtypeStruct((_H, _D, _B), jnp.float32),
    scratch_types=[
        pltpu.VMEM((_IROWS, _IDXW), jnp.int32),
        pltpu.VMEM((2, _CHUNK, _D), jnp.float32),
        pltpu.VMEM((2, _D, _CHUNK), jnp.float32),
        pltpu.SemaphoreType.DMA,
        pltpu.SemaphoreType.DMA,
        pltpu.SemaphoreType.DMA,
        pltpu.SemaphoreType.DMA,
    ],
    compiler_params=pltpu.CompilerParams(
        use_tc_tiling_on_sc=False, needs_layout_passes=False
    ),
)
def _gather_t(idx_hbm, table_hbm, out_hbm, idx_v, rows_v, rowst_v,
              gsem0, gsem1, osem0, osem1):
    wid = lax.axis_index("s") * _NC + lax.axis_index("c")
    irow0 = pl.multiple_of(wid * _IROWS, 8)
    pltpu.sync_copy(idx_hbm.at[pl.ds(irow0, _IROWS)], idx_v)
    gsems = (gsem0, gsem1)
    osems = (osem0, osem1)

    def gather_copies(k, x, make):
        mk = pltpu.make_async_copy if make else pltpu.async_copy
        return [
            mk(
                table_hbm.at[idx_v.at[k * _NSUB + j]],
                rows_v.at[x, pl.ds(j * _IDXW, _IDXW)],
                gsems[x],
            )
            for j in range(_NSUB)
        ]

    def out_copy(k, x, make):
        mk = pltpu.make_async_copy if make else pltpu.async_copy
        flat = wid * _NPW + k * _CHUNK
        h = flat // _B
        b0 = pl.multiple_of(flat % _B, _CHUNK)
        return mk(
            rowst_v.at[x],
            out_hbm.at[h, pl.ds(0, _D), pl.ds(b0, _CHUNK)],
            osems[x],
        )

    def transpose(x):
        @pl.loop(0, _CHUNK // 16)
        def _txp(j):
            lane16 = lax.iota(jnp.int32, 16)
            row16 = j * 16 + lane16
            for e in range(_D):
                col16 = (e + lane16) & (_D - 1)
                vals = plsc.load_gather(rows_v.at[x], [row16, col16])
                plsc.store_scatter(rowst_v.at[x], [col16, row16], vals)

    def step(k, x, first, last):
        if not last:

            @pl.when(k + 1 < _NTASK)
            def _():
                gather_copies(k + 1, 1 - x, make=False)
        for cp in gather_copies(k, x, make=True):
            cp.wait()
        if not first:

            @pl.when(k >= 2)
            def _():
                out_copy(k - 2, x, make=True).wait()

        transpose(x)
        out_copy(k, x, make=False)

    gather_copies(0, 0, make=False)

    @pl.loop(0, _NTASK // 2)
    def _pair(gp):
        k0 = gp * 2
        step(k0, 0, first=False, last=False)
        step(k0 + 1, 1, first=False, last=False)

    out_copy(_NTASK - 2, 0, make=True).wait()
    out_copy(_NTASK - 1, 1, make=True).wait()


def kernel(d, table):
    idx = d.T.reshape(_N // _IDXW, _IDXW).astype(jnp.int32)
    out_t = _gather_t(idx, table)
    return jnp.transpose(out_t, (2, 0, 1))


# speedup vs baseline: 2.2696x; 1.0084x over previous
import functools

import jax
import jax.numpy as jnp
from jax import lax
from jax.experimental import pallas as pl
from jax.experimental.pallas import tpu as pltpu
from jax.experimental.pallas import tpu_sc as plsc

_B = 16384
_H = 50
_D = 32
_N = _B * _H
_NC = 2
_NS = 16
_NW = _NC * _NS
_CHUNK = 512
_NSUB = _CHUNK // 128
_NT = _N // _CHUNK
_TPW = _NT // _NW
_NTILED = (_H // 8) * 128 * 2
_IROWS = _TPW * _NSUB

_mesh = plsc.VectorSubcoreMesh(core_axis_name="c", subcore_axis_name="s")


@functools.partial(
    pl.kernel,
    mesh=_mesh,
    out_type=jax.ShapeDtypeStruct((_H, _D, _B), jnp.float32),
    scratch_types=[
        pltpu.VMEM((_IROWS, 128), jnp.int32),
        pltpu.VMEM((2, _CHUNK, _D), jnp.float32),
        pltpu.VMEM((2, 4, _D, 128), jnp.float32),
        pltpu.VMEM((2, _D, _CHUNK), jnp.float32),
        pltpu.SemaphoreType.DMA,
        pltpu.SemaphoreType.DMA,
        pltpu.SemaphoreType.DMA,
        pltpu.SemaphoreType.DMA,
    ],
    compiler_params=pltpu.CompilerParams(
        use_tc_tiling_on_sc=False, needs_layout_passes=False
    ),
)
def _gather_t(idxa_hbm, idxb_hbm, table_hbm, out_hbm, idx_v, rows_v,
              rowstt_v, rowstl_v, gsem0, gsem1, osem0, osem1):
    wid = lax.axis_index("s") * _NC + lax.axis_index("c")
    gsems = (gsem0, gsem1)
    osems = (osem0, osem1)

    na = 6144

    @pl.when(wid < 30)
    def _():
        r0 = pl.multiple_of(wid * _IROWS, 8)
        pltpu.sync_copy(idxa_hbm.at[pl.ds(r0, _IROWS)], idx_v)

    @pl.when(wid == 30)
    def _():
        pltpu.sync_copy(idxa_hbm.at[pl.ds(na - 144, 144)],
                        idx_v.at[pl.ds(0, 144)])
        pltpu.sync_copy(idxb_hbm.at[pl.ds(0, 56)],
                        idx_v.at[pl.ds(144, 56)])

    @pl.when(wid == 31)
    def _():
        pltpu.sync_copy(idxb_hbm.at[pl.ds(56, 200)], idx_v)

    def decode(k):
        t = wid * _TPW + k
        tiled = t < _NTILED
        tr = t // 256
        rem = t % 256
        h0 = tr * 8 + (rem % 2) * 4
        b0t = pl.multiple_of((rem // 2) * 128, 128)
        tl = t - _NTILED
        hl = 48 + tl // (_B // _CHUNK)
        b0l = pl.multiple_of((tl % (_B // _CHUNK)) * _CHUNK, _CHUNK)
        return tiled, h0, b0t, hl, b0l

    def gather_copies(k, x, make):
        mk = pltpu.make_async_copy if make else pltpu.async_copy
        return [
            mk(
                table_hbm.at[idx_v.at[k * _NSUB + j]],
                rows_v.at[x, pl.ds(j * 128, 128)],
                gsems[x],
            )
            for j in range(_NSUB)
        ]

    def out_copies(k, x, make, wait):
        mk = pltpu.make_async_copy if make else pltpu.async_copy
        tiled, h0, b0t, hl, b0l = decode(k)

        @pl.when(tiled)
        def _():
            cp = mk(
                rowstt_v.at[x],
                out_hbm.at[pl.ds(h0, 4), pl.ds(0, _D), pl.ds(b0t, 128)],
                osems[x],
            )
            if wait:
                cp.wait()

        @pl.when(jnp.logical_not(tiled))
        def _():
            cp = mk(
                rowstl_v.at[x],
                out_hbm.at[hl, pl.ds(0, _D), pl.ds(b0l, _CHUNK)],
                osems[x],
            )
            if wait:
                cp.wait()

    def transpose(k, x):
        tiled, *_ = decode(k)

        @pl.when(tiled)
        def _():
            @pl.loop(0, _CHUNK // 16)
            def _txp(j):
                lane16 = lax.iota(jnp.int32, 16)
                i16 = j * 16 + lane16
                r16 = jnp.full((16,), j // 8, jnp.int32)
                c16 = (j % 8) * 16 + lane16
                for e in range(_D):
                    e16 = (e + lane16) & (_D - 1)
                    vals = plsc.load_gather(rows_v.at[x], [i16, e16])
                    plsc.store_scatter(rowstt_v.at[x], [r16, e16, c16], vals)

        @pl.when(jnp.logical_not(tiled))
        def _():
            @pl.loop(0, _CHUNK // 16)
            def _txp(j):
                lane16 = lax.iota(jnp.int32, 16)
                i16 = j * 16 + lane16
                for e in range(_D):
                    e16 = (e + lane16) & (_D - 1)
                    vals = plsc.load_gather(rows_v.at[x], [i16, e16])
                    plsc.store_scatter(rowstl_v.at[x], [e16, i16], vals)

    def step(k, x):
        @pl.when(k + 1 < _TPW)
        def _():
            gather_copies(k + 1, 1 - x, make=False)

        for cp in gather_copies(k, x, make=True):
            cp.wait()

        @pl.when(k >= 2)
        def _():
            out_copies(k - 2, x, make=True, wait=True)

        transpose(k, x)
        out_copies(k, x, make=False, wait=False)

    gather_copies(0, 0, make=False)

    @pl.loop(0, _TPW // 2)
    def _pair(gp):
        step(gp * 2, 0)
        step(gp * 2 + 1, 1)

    out_copies(_TPW - 2, 0, make=True, wait=True)
    out_copies(_TPW - 1, 1, make=True, wait=True)


def kernel(d, table):
    dt = d.T.astype(jnp.int32)
    idxa = (
        lax.slice(dt, (0, 0), (48, _B))
        .reshape(6, 8, 128, 128)
        .transpose(0, 2, 1, 3)
        .reshape(6144, 128)
    )
    idxb = lax.slice(dt, (48, 0), (_H, _B)).reshape(256, 128)
    out_t = _gather_t(idxa, idxb, table)
    return jnp.transpose(out_t, (2, 0, 1))
